# baseline (speedup 1.0000x reference)
import functools

import jax
import jax.numpy as jnp
from jax import lax
from jax.experimental import pallas as pl
from jax.experimental.pallas import tpu as pltpu
from jax.experimental.pallas import tpu_sc as plsc

_NUM_ENTITY = 1000000
_DIM = 32
_BATCH = 16384
_EPR = 128 // _DIM
_NROWS = _NUM_ENTITY // _EPR
_NCOL = _NUM_ENTITY // 128
_TAIL = _NUM_ENTITY - _NCOL * 128

_info = plsc.get_sparse_core_info()
_NC = _info.num_cores
_NS = _info.num_subcores
_L = _info.num_lanes
_NW = _NC * _NS
_CPW = (_NCOL + 1 + _NW - 1) // _NW

_BPW = _BATCH // _NW
_HALF = _BPW // 2
_CHUNK = 128
_GROUPS = _HALF // _L

_mesh = plsc.VectorSubcoreMesh(core_axis_name="c", subcore_axis_name="s")
_params = pltpu.CompilerParams(
    needs_layout_passes=False, use_tc_tiling_on_sc=True)


def _shuffle(src, dst, nrow, width):
    lanes = lax.iota(jnp.int32, _L)
    for g8 in range(8):
        d_vec = (g8 & 1) * 16 + lanes
        for j in range(nrow):
            col = 4 * j + (g8 >> 1)
            if col >= width:
                continue
            v = plsc.load_gather(src, [d_vec, jnp.full((_L,), col, jnp.int32)])
            dst[j, pl.ds(g8 * _L, _L)] = v


@functools.partial(
    pl.kernel,
    mesh=_mesh,
    compiler_params=_params,
    out_type=(jax.ShapeDtypeStruct((_NROWS, 128), jnp.float32),
              jax.ShapeDtypeStruct((_NROWS, 128), jnp.float32)),
    scratch_types=[
        pltpu.VMEM((32, 128), jnp.float32),
        pltpu.VMEM((32, 128), jnp.float32),
        pltpu.VMEM((32, 128), jnp.float32),
        pltpu.VMEM((32, 128), jnp.float32),
        pltpu.VMEM((32, 128), jnp.float32),
        pltpu.VMEM((32, 128), jnp.float32),
        pltpu.VMEM((32, 128), jnp.float32),
        pltpu.VMEM((32, 128), jnp.float32),
        pltpu.SemaphoreType.DMA,
        pltpu.SemaphoreType.DMA,
        pltpu.SemaphoreType.DMA,
    ],
)
def _sc_relayout(wt_in, wt_out, tail_in, tail_out, g_in, g_out,
                 ia0, ia1, ib0, ib1, oa0, oa1, ob0, ob1,
                 sin_a, sin_b, sout):
    wid = lax.axis_index("s") * _NC + lax.axis_index("c")
    lo = wid * _CPW
    hi = jnp.minimum(lo + _CPW, _NCOL)
    n = hi - lo
    ins = ((ia0, ia1), (ib0, ib1))
    outs = ((oa0, oa1), (ob0, ob1))
    sems = (sin_a, sin_b)

    def fetch(q, buf):
        sl = pl.ds(pl.multiple_of(q * 128, 128), 128)
        pltpu.async_copy(wt_in.at[:, sl], ins[buf][0], sems[buf])
        pltpu.async_copy(wt_out.at[:, sl], ins[buf][1], sems[buf])

    def drain_in(buf):
        pltpu.make_async_copy(
            wt_in.at[:, pl.ds(0, 128)], ins[buf][0], sems[buf]).wait()
        pltpu.make_async_copy(
            wt_in.at[:, pl.ds(0, 128)], ins[buf][1], sems[buf]).wait()

    def drain_out():
        pltpu.make_async_copy(
            wt_in.at[:, pl.ds(0, 128)], oa0, sout).wait()
        pltpu.make_async_copy(
            wt_in.at[:, pl.ds(0, 128)], oa1, sout).wait()

    @pl.when(n > 0)
    def _():
        fetch(lo, 0)

        def body(c, carry):
            q = lo + c
            even = lax.rem(c, 2) == 0

            @pl.when(c + 1 < n)
            def _():
                @pl.when(even)
                def _():
                    fetch(q + 1, 1)

                @pl.when(jnp.logical_not(even))
                def _():
                    fetch(q + 1, 0)

            @pl.when(c >= 2)
            def _():
                drain_out()

            @pl.when(even)
            def _():
                drain_in(0)
                _shuffle(ia0, oa0, 32, 128)
                _shuffle(ia1, oa1, 32, 128)
                pltpu.async_copy(oa0, g_in.at[pl.ds(pl.multiple_of(q * 32, 32), 32)], sout)
                pltpu.async_copy(oa1, g_out.at[pl.ds(pl.multiple_of(q * 32, 32), 32)], sout)

            @pl.when(jnp.logical_not(even))
            def _():
                drain_in(1)
                _shuffle(ib0, ob0, 32, 128)
                _shuffle(ib1, ob1, 32, 128)
                pltpu.async_copy(ob0, g_in.at[pl.ds(pl.multiple_of(q * 32, 32), 32)], sout)
                pltpu.async_copy(ob1, g_out.at[pl.ds(pl.multiple_of(q * 32, 32), 32)], sout)

            return carry

        lax.fori_loop(0, n, body, 0)

        @pl.when(n >= 2)
        def _():
            drain_out()

        @pl.when(n >= 1)
        def _():
            drain_out()

    @pl.when(wid == _NW - 1)
    def _():
        nr = _TAIL // 4
        rows = pl.ds(_NCOL * 32, nr)
        pltpu.sync_copy(tail_in, ia0.at[pl.ds(0, nr)])
        pltpu.sync_copy(tail_out, ia1.at[pl.ds(0, nr)])
        pltpu.sync_copy(ia0.at[pl.ds(0, nr)], g_in.at[rows])
        pltpu.sync_copy(ia1.at[pl.ds(0, nr)], g_out.at[rows])


@functools.partial(
    pl.kernel,
    mesh=_mesh,
    compiler_params=_params,
    out_type=jax.ShapeDtypeStruct((_BATCH,), jnp.float32),
    scratch_types=[
        pltpu.VMEM((_BPW,), jnp.int32),
        pltpu.VMEM((_BPW,), jnp.int32),
        pltpu.VMEM((_HALF, 128), jnp.float32),
        pltpu.VMEM((_HALF, 128), jnp.float32),
        pltpu.VMEM((_HALF,), jnp.int32),
        pltpu.VMEM((_HALF,), jnp.int32),
        pltpu.VMEM((_BPW,), jnp.float32),
        pltpu.SemaphoreType.DMA,
        pltpu.SemaphoreType.DMA,
    ],
)
def _sc_pair_dot(idx0_hbm, idx1_hbm, win_hbm, wout_hbm, out_hbm,
                 idx0_v, idx1_v, in_bl, out_bl, row0_v, row1_v, res_v,
                 sem_a, sem_b):
    wid = lax.axis_index("s") * _NC + lax.axis_index("c")
    base = wid * _BPW

    pltpu.sync_copy(idx0_hbm.at[pl.ds(base, _BPW)], idx0_v)
    pltpu.sync_copy(idx1_hbm.at[pl.ds(base, _BPW)], idx1_v)

    lanes = lax.iota(jnp.int32, _L)

    for half in range(2):
        hoff = half * _HALF

        def rows_body(g, carry):
            sl = pl.ds(g * _L, _L)
            row0_v[sl] = lax.shift_right_logical(
                idx0_v[pl.ds(hoff + g * _L, _L)], 2)
            row1_v[sl] = lax.shift_right_logical(
                idx1_v[pl.ds(hoff + g * _L, _L)], 2)
            return carry

        lax.fori_loop(0, _GROUPS, rows_body, 0)

        copies = []
        for k in range(_HALF // _CHUNK):
            sl = pl.ds(k * _CHUNK, _CHUNK)
            copies.append(pltpu.async_copy(
                win_hbm.at[row0_v.at[sl]], in_bl.at[sl], sem_a))
            copies.append(pltpu.async_copy(
                wout_hbm.at[row1_v.at[sl]], out_bl.at[sl], sem_b))
        for cp in copies:
            cp.wait()

        def dot_body(g, carry):
            i_vec = g * _L + lanes
            a0 = lax.bitwise_and(idx0_v[pl.ds(hoff + g * _L, _L)], _EPR - 1)
            a1 = lax.bitwise_and(idx1_v[pl.ds(hoff + g * _L, _L)], _EPR - 1)
            col0 = a0 * _DIM
            col1 = a1 * _DIM
            acc = jnp.zeros((_L,), jnp.float32)
            for d in range(_DIM):
                va = plsc.load_gather(in_bl, [i_vec, col0 + d])
                vb = plsc.load_gather(out_bl, [i_vec, col1 + d])
                acc = acc + va * vb
            res_v[pl.ds(hoff + g * _L, _L)] = acc
            return carry

        lax.fori_loop(0, _GROUPS, dot_body, 0)

    pltpu.sync_copy(res_v, out_hbm.at[pl.ds(base, _BPW)])


def kernel(idxs, W_in, W_out):
    idx0 = idxs[:, 0].astype(jnp.int32)
    idx1 = idxs[:, 1].astype(jnp.int32)
    tail_in = W_in[_NCOL * 128:].reshape(_TAIL // 4, 128)
    tail_out = W_out[_NCOL * 128:].reshape(_TAIL // 4, 128)
    g_in, g_out = _sc_relayout(W_in.T, W_out.T, tail_in, tail_out)
    return _sc_pair_dot(idx0, idx1, g_in, g_out)


# speedup vs baseline: 3.1451x; 3.1451x over previous
import functools

import jax
import jax.numpy as jnp
from jax import lax
from jax.experimental import pallas as pl
from jax.experimental.pallas import tpu as pltpu
from jax.experimental.pallas import tpu_sc as plsc

_NUM_ENTITY = 1000000
_DIM = 32
_BATCH = 16384
_EPR = 128 // _DIM
_NROWS = _NUM_ENTITY // _EPR
_NCOL = _NUM_ENTITY // 128
_TAIL = _NUM_ENTITY - _NCOL * 128

_info = plsc.get_sparse_core_info()
_NC = _info.num_cores
_NS = _info.num_subcores
_L = _info.num_lanes
_NW = _NC * _NS
_CPW = (_NCOL + 1 + _NW - 1) // _NW

_BPW = _BATCH // _NW
_HALF = _BPW // 2
_CHUNK = 128
_GROUPS = _HALF // _L

_mesh = plsc.VectorSubcoreMesh(core_axis_name="c", subcore_axis_name="s")
_params = pltpu.CompilerParams(
    needs_layout_passes=False, use_tc_tiling_on_sc=True)


def _shuffle2(src0, dst0, src1, dst1):
    lanes = lax.iota(jnp.int32, _L)

    def body(t, carry):
        be = lax.shift_right_logical(t, 4)
        k = lax.bitwise_and(t, _L - 1)
        e_vec = be * _L + lax.bitwise_and(lanes + k, _L - 1)
        j_vec = lax.shift_right_logical(e_vec, 2)
        a32 = lax.bitwise_and(e_vec, 3) * _DIM
        for bd in range(2):
            d_vec = bd * _L + lanes
            l_vec = a32 + d_vec
            v0 = plsc.load_gather(src0, [d_vec, e_vec])
            plsc.store_scatter(dst0, [j_vec, l_vec], v0)
            v1 = plsc.load_gather(src1, [d_vec, e_vec])
            plsc.store_scatter(dst1, [j_vec, l_vec], v1)
        return carry

    lax.fori_loop(0, 8 * _L, body, 0)


@functools.partial(
    pl.kernel,
    mesh=_mesh,
    compiler_params=_params,
    out_type=(jax.ShapeDtypeStruct((_NROWS, 128), jnp.float32),
              jax.ShapeDtypeStruct((_NROWS, 128), jnp.float32)),
    scratch_types=[
        pltpu.VMEM((32, 128), jnp.float32),
        pltpu.VMEM((32, 128), jnp.float32),
        pltpu.VMEM((32, 128), jnp.float32),
        pltpu.VMEM((32, 128), jnp.float32),
        pltpu.VMEM((32, 128), jnp.float32),
        pltpu.VMEM((32, 128), jnp.float32),
        pltpu.VMEM((32, 128), jnp.float32),
        pltpu.VMEM((32, 128), jnp.float32),
        pltpu.SemaphoreType.DMA,
        pltpu.SemaphoreType.DMA,
        pltpu.SemaphoreType.DMA,
    ],
)
def _sc_relayout(wt_in, wt_out, tail_in, tail_out, g_in, g_out,
                 ia0, ia1, ib0, ib1, oa0, oa1, ob0, ob1,
                 sin_a, sin_b, sout):
    wid = lax.axis_index("s") * _NC + lax.axis_index("c")
    lo = wid * _CPW
    hi = jnp.minimum(lo + _CPW, _NCOL)
    n = hi - lo
    ins = ((ia0, ia1), (ib0, ib1))
    outs = ((oa0, oa1), (ob0, ob1))
    sems = (sin_a, sin_b)

    def fetch(q, buf):
        sl = pl.ds(pl.multiple_of(q * 128, 128), 128)
        pltpu.async_copy(wt_in.at[:, sl], ins[buf][0], sems[buf])
        pltpu.async_copy(wt_out.at[:, sl], ins[buf][1], sems[buf])

    def drain_in(buf):
        pltpu.make_async_copy(
            wt_in.at[:, pl.ds(0, 128)], ins[buf][0], sems[buf]).wait()
        pltpu.make_async_copy(
            wt_in.at[:, pl.ds(0, 128)], ins[buf][1], sems[buf]).wait()

    def drain_out():
        pltpu.make_async_copy(
            wt_in.at[:, pl.ds(0, 128)], oa0, sout).wait()
        pltpu.make_async_copy(
            wt_in.at[:, pl.ds(0, 128)], oa1, sout).wait()

    @pl.when(n > 0)
    def _():
        fetch(lo, 0)

        def body(c, carry):
            q = lo + c
            even = lax.rem(c, 2) == 0

            @pl.when(c + 1 < n)
            def _():
                @pl.when(even)
                def _():
                    fetch(q + 1, 1)

                @pl.when(jnp.logical_not(even))
                def _():
                    fetch(q + 1, 0)

            @pl.when(c >= 2)
            def _():
                drain_out()

            @pl.when(even)
            def _():
                drain_in(0)
                _shuffle2(ia0, oa0, ia1, oa1)
                pltpu.async_copy(oa0, g_in.at[pl.ds(pl.multiple_of(q * 32, 32), 32)], sout)
                pltpu.async_copy(oa1, g_out.at[pl.ds(pl.multiple_of(q * 32, 32), 32)], sout)

            @pl.when(jnp.logical_not(even))
            def _():
                drain_in(1)
                _shuffle2(ib0, ob0, ib1, ob1)
                pltpu.async_copy(ob0, g_in.at[pl.ds(pl.multiple_of(q * 32, 32), 32)], sout)
                pltpu.async_copy(ob1, g_out.at[pl.ds(pl.multiple_of(q * 32, 32), 32)], sout)

            return carry

        lax.fori_loop(0, n, body, 0)

        @pl.when(n >= 2)
        def _():
            drain_out()

        @pl.when(n >= 1)
        def _():
            drain_out()

    @pl.when(wid == _NW - 1)
    def _():
        nr = _TAIL // 4
        rows = pl.ds(_NCOL * 32, nr)
        pltpu.sync_copy(tail_in, ia0.at[pl.ds(0, nr)])
        pltpu.sync_copy(tail_out, ia1.at[pl.ds(0, nr)])
        pltpu.sync_copy(ia0.at[pl.ds(0, nr)], g_in.at[rows])
        pltpu.sync_copy(ia1.at[pl.ds(0, nr)], g_out.at[rows])


@functools.partial(
    pl.kernel,
    mesh=_mesh,
    compiler_params=_params,
    out_type=jax.ShapeDtypeStruct((_BATCH,), jnp.float32),
    scratch_types=[
        pltpu.VMEM((_BPW,), jnp.int32),
        pltpu.VMEM((_BPW,), jnp.int32),
        pltpu.VMEM((_HALF, 128), jnp.float32),
        pltpu.VMEM((_HALF, 128), jnp.float32),
        pltpu.VMEM((_HALF,), jnp.int32),
        pltpu.VMEM((_HALF,), jnp.int32),
        pltpu.VMEM((_BPW,), jnp.float32),
        pltpu.SemaphoreType.DMA,
        pltpu.SemaphoreType.DMA,
    ],
)
def _sc_pair_dot(idx0_hbm, idx1_hbm, win_hbm, wout_hbm, out_hbm,
                 idx0_v, idx1_v, in_bl, out_bl, row0_v, row1_v, res_v,
                 sem_a, sem_b):
    wid = lax.axis_index("s") * _NC + lax.axis_index("c")
    base = wid * _BPW

    pltpu.sync_copy(idx0_hbm.at[pl.ds(base, _BPW)], idx0_v)
    pltpu.sync_copy(idx1_hbm.at[pl.ds(base, _BPW)], idx1_v)

    lanes = lax.iota(jnp.int32, _L)

    for half in range(2):
        hoff = half * _HALF

        def rows_body(g, carry):
            sl = pl.ds(g * _L, _L)
            row0_v[sl] = lax.shift_right_logical(
                idx0_v[pl.ds(hoff + g * _L, _L)], 2)
            row1_v[sl] = lax.shift_right_logical(
                idx1_v[pl.ds(hoff + g * _L, _L)], 2)
            return carry

        lax.fori_loop(0, _GROUPS, rows_body, 0)

        copies = []
        for k in range(_HALF // _CHUNK):
            sl = pl.ds(k * _CHUNK, _CHUNK)
            copies.append(pltpu.async_copy(
                win_hbm.at[row0_v.at[sl]], in_bl.at[sl], sem_a))
            copies.append(pltpu.async_copy(
                wout_hbm.at[row1_v.at[sl]], out_bl.at[sl], sem_b))
        for cp in copies:
            cp.wait()

        def dot_body(g, carry):
            i_vec = g * _L + lanes
            a0 = lax.bitwise_and(idx0_v[pl.ds(hoff + g * _L, _L)], _EPR - 1)
            a1 = lax.bitwise_and(idx1_v[pl.ds(hoff + g * _L, _L)], _EPR - 1)
            col0 = a0 * _DIM
            col1 = a1 * _DIM
            acc = jnp.zeros((_L,), jnp.float32)
            for d in range(_DIM):
                dvar = lax.bitwise_and(lanes + d, _DIM - 1)
                va = plsc.load_gather(in_bl, [i_vec, col0 + dvar])
                vb = plsc.load_gather(out_bl, [i_vec, col1 + dvar])
                acc = acc + va * vb
            res_v[pl.ds(hoff + g * _L, _L)] = acc
            return carry

        lax.fori_loop(0, _GROUPS, dot_body, 0)

    pltpu.sync_copy(res_v, out_hbm.at[pl.ds(base, _BPW)])


def kernel(idxs, W_in, W_out):
    idx0 = idxs[:, 0].astype(jnp.int32)
    idx1 = idxs[:, 1].astype(jnp.int32)
    tail_in = W_in[_NCOL * 128:].reshape(_TAIL // 4, 128)
    tail_out = W_out[_NCOL * 128:].reshape(_TAIL // 4, 128)
    g_in, g_out = _sc_relayout(W_in.T, W_out.T, tail_in, tail_out)
    return _sc_pair_dot(idx0, idx1, g_in, g_out)


# speedup vs baseline: 3.3211x; 1.0560x over previous
import functools

import jax
import jax.numpy as jnp
from jax import lax
from jax.experimental import pallas as pl
from jax.experimental.pallas import tpu as pltpu
from jax.experimental.pallas import tpu_sc as plsc

_NUM_ENTITY = 1000000
_DIM = 32
_BATCH = 16384
_EPR = 128 // _DIM
_NROWS = _NUM_ENTITY // _EPR
_NCOL = _NUM_ENTITY // 128
_TAIL = _NUM_ENTITY - _NCOL * 128

_info = plsc.get_sparse_core_info()
_NC = _info.num_cores
_NS = _info.num_subcores
_L = _info.num_lanes
_NW = _NC * _NS
_CPW = (_NCOL + 1 + _NW - 1) // _NW

_BPW = _BATCH // _NW
_HALF = _BPW // 2
_CHUNK = 128
_GROUPS = _HALF // _L

_mesh = plsc.VectorSubcoreMesh(core_axis_name="c", subcore_axis_name="s")
_params = pltpu.CompilerParams(
    needs_layout_passes=False, use_tc_tiling_on_sc=True)


def _shuffle2(src0, dst0, src1, dst1):
    lanes = lax.iota(jnp.int32, _L)

    def body(t, carry):
        for u in range(4):
            tu = t * 4 + u
            be = lax.shift_right_logical(tu, 4)
            k = lax.bitwise_and(tu, _L - 1)
            e_vec = be * _L + lax.bitwise_and(lanes + k, _L - 1)
            j_vec = lax.shift_right_logical(e_vec, 2)
            a32 = lax.bitwise_and(e_vec, 3) * _DIM
            for bd in range(2):
                d_vec = bd * _L + lanes
                l_vec = a32 + d_vec
                v0 = plsc.load_gather(src0, [d_vec, e_vec])
                plsc.store_scatter(dst0, [j_vec, l_vec], v0)
                v1 = plsc.load_gather(src1, [d_vec, e_vec])
                plsc.store_scatter(dst1, [j_vec, l_vec], v1)
        return carry

    lax.fori_loop(0, 2 * _L, body, 0)


@functools.partial(
    pl.kernel,
    mesh=_mesh,
    compiler_params=_params,
    out_type=(jax.ShapeDtypeStruct((_NROWS, 128), jnp.float32),
              jax.ShapeDtypeStruct((_NROWS, 128), jnp.float32)),
    scratch_types=[
        pltpu.VMEM((32, 128), jnp.float32),
        pltpu.VMEM((32, 128), jnp.float32),
        pltpu.VMEM((32, 128), jnp.float32),
        pltpu.VMEM((32, 128), jnp.float32),
        pltpu.VMEM((32, 128), jnp.float32),
        pltpu.VMEM((32, 128), jnp.float32),
        pltpu.VMEM((32, 128), jnp.float32),
        pltpu.VMEM((32, 128), jnp.float32),
        pltpu.SemaphoreType.DMA,
        pltpu.SemaphoreType.DMA,
        pltpu.SemaphoreType.DMA,
    ],
)
def _sc_relayout(wt_in, wt_out, tail_in, tail_out, g_in, g_out,
                 ia0, ia1, ib0, ib1, oa0, oa1, ob0, ob1,
                 sin_a, sin_b, sout):
    wid = lax.axis_index("s") * _NC + lax.axis_index("c")
    lo = wid * _CPW
    hi = jnp.minimum(lo + _CPW, _NCOL)
    n = hi - lo
    ins = ((ia0, ia1), (ib0, ib1))
    outs = ((oa0, oa1), (ob0, ob1))
    sems = (sin_a, sin_b)

    def fetch(q, buf):
        sl = pl.ds(pl.multiple_of(q * 128, 128), 128)
        pltpu.async_copy(wt_in.at[:, sl], ins[buf][0], sems[buf])
        pltpu.async_copy(wt_out.at[:, sl], ins[buf][1], sems[buf])

    def drain_in(buf):
        pltpu.make_async_copy(
            wt_in.at[:, pl.ds(0, 128)], ins[buf][0], sems[buf]).wait()
        pltpu.make_async_copy(
            wt_in.at[:, pl.ds(0, 128)], ins[buf][1], sems[buf]).wait()

    def drain_out():
        pltpu.make_async_copy(
            wt_in.at[:, pl.ds(0, 128)], oa0, sout).wait()
        pltpu.make_async_copy(
            wt_in.at[:, pl.ds(0, 128)], oa1, sout).wait()

    @pl.when(n > 0)
    def _():
        fetch(lo, 0)

        def body(c, carry):
            q = lo + c
            even = lax.rem(c, 2) == 0

            @pl.when(c + 1 < n)
            def _():
                @pl.when(even)
                def _():
                    fetch(q + 1, 1)

                @pl.when(jnp.logical_not(even))
                def _():
                    fetch(q + 1, 0)

            @pl.when(c >= 2)
            def _():
                drain_out()

            @pl.when(even)
            def _():
                drain_in(0)
                _shuffle2(ia0, oa0, ia1, oa1)
                pltpu.async_copy(oa0, g_in.at[pl.ds(pl.multiple_of(q * 32, 32), 32)], sout)
                pltpu.async_copy(oa1, g_out.at[pl.ds(pl.multiple_of(q * 32, 32), 32)], sout)

            @pl.when(jnp.logical_not(even))
            def _():
                drain_in(1)
                _shuffle2(ib0, ob0, ib1, ob1)
                pltpu.async_copy(ob0, g_in.at[pl.ds(pl.multiple_of(q * 32, 32), 32)], sout)
                pltpu.async_copy(ob1, g_out.at[pl.ds(pl.multiple_of(q * 32, 32), 32)], sout)

            return carry

        lax.fori_loop(0, n, body, 0)

        @pl.when(n >= 2)
        def _():
            drain_out()

        @pl.when(n >= 1)
        def _():
            drain_out()

    @pl.when(wid == _NW - 1)
    def _():
        nr = _TAIL // 4
        rows = pl.ds(_NCOL * 32, nr)
        pltpu.sync_copy(tail_in, ia0.at[pl.ds(0, nr)])
        pltpu.sync_copy(tail_out, ia1.at[pl.ds(0, nr)])
        pltpu.sync_copy(ia0.at[pl.ds(0, nr)], g_in.at[rows])
        pltpu.sync_copy(ia1.at[pl.ds(0, nr)], g_out.at[rows])


@functools.partial(
    pl.kernel,
    mesh=_mesh,
    compiler_params=_params,
    out_type=jax.ShapeDtypeStruct((_BATCH,), jnp.float32),
    scratch_types=[
        pltpu.VMEM((_BPW,), jnp.int32),
        pltpu.VMEM((_BPW,), jnp.int32),
        pltpu.VMEM((_HALF, 128), jnp.float32),
        pltpu.VMEM((_HALF, 128), jnp.float32),
        pltpu.VMEM((_HALF,), jnp.int32),
        pltpu.VMEM((_HALF,), jnp.int32),
        pltpu.VMEM((_BPW,), jnp.float32),
        pltpu.SemaphoreType.DMA,
        pltpu.SemaphoreType.DMA,
    ],
)
def _sc_pair_dot(idx0_hbm, idx1_hbm, win_hbm, wout_hbm, out_hbm,
                 idx0_v, idx1_v, in_bl, out_bl, row0_v, row1_v, res_v,
                 sem_a, sem_b):
    wid = lax.axis_index("s") * _NC + lax.axis_index("c")
    base = wid * _BPW

    pltpu.sync_copy(idx0_hbm.at[pl.ds(base, _BPW)], idx0_v)
    pltpu.sync_copy(idx1_hbm.at[pl.ds(base, _BPW)], idx1_v)

    lanes = lax.iota(jnp.int32, _L)

    for half in range(2):
        hoff = half * _HALF

        def rows_body(g, carry):
            sl = pl.ds(g * _L, _L)
            row0_v[sl] = lax.shift_right_logical(
                idx0_v[pl.ds(hoff + g * _L, _L)], 2)
            row1_v[sl] = lax.shift_right_logical(
                idx1_v[pl.ds(hoff + g * _L, _L)], 2)
            return carry

        lax.fori_loop(0, _GROUPS, rows_body, 0)

        copies = []
        for k in range(_HALF // _CHUNK):
            sl = pl.ds(k * _CHUNK, _CHUNK)
            copies.append(pltpu.async_copy(
                win_hbm.at[row0_v.at[sl]], in_bl.at[sl], sem_a))
            copies.append(pltpu.async_copy(
                wout_hbm.at[row1_v.at[sl]], out_bl.at[sl], sem_b))
        for cp in copies:
            cp.wait()

        def dot_body(g, carry):
            i_vec = g * _L + lanes
            a0 = lax.bitwise_and(idx0_v[pl.ds(hoff + g * _L, _L)], _EPR - 1)
            a1 = lax.bitwise_and(idx1_v[pl.ds(hoff + g * _L, _L)], _EPR - 1)
            col0 = a0 * _DIM
            col1 = a1 * _DIM
            acc = jnp.zeros((_L,), jnp.float32)
            for d in range(_DIM):
                dvar = lax.bitwise_and(lanes + d, _DIM - 1)
                va = plsc.load_gather(in_bl, [i_vec, col0 + dvar])
                vb = plsc.load_gather(out_bl, [i_vec, col1 + dvar])
                acc = acc + va * vb
            res_v[pl.ds(hoff + g * _L, _L)] = acc
            return carry

        lax.fori_loop(0, _GROUPS, dot_body, 0)

    pltpu.sync_copy(res_v, out_hbm.at[pl.ds(base, _BPW)])


def kernel(idxs, W_in, W_out):
    idx0 = idxs[:, 0].astype(jnp.int32)
    idx1 = idxs[:, 1].astype(jnp.int32)
    tail_in = W_in[_NCOL * 128:].reshape(_TAIL // 4, 128)
    tail_out = W_out[_NCOL * 128:].reshape(_TAIL // 4, 128)
    g_in, g_out = _sc_relayout(W_in.T, W_out.T, tail_in, tail_out)
    return _sc_pair_dot(idx0, idx1, g_in, g_out)
